# baseline (speedup 1.0000x reference)
import functools

import jax
import jax.numpy as jnp
from jax import lax
from jax.experimental import pallas as pl
from jax.experimental.pallas import tpu as pltpu
from jax.experimental.pallas import tpu_sc as plsc

_B = 128
_V = 100000
_NC = 2
_NS = 16
_NW = _NC * _NS
_L = 16
_NG = _B // _L

_KTC = 40000
_BKTC = 2000
_GTC = _KTC // _BKTC

_T0 = _KTC // 8
_NT = _V // 8
_WT = 236
_CHT = 59
_CHCLS = _CHT * 8
_NCH = _WT // _CHT
_IMAX = 2**31 - 1

_mesh = plsc.VectorSubcoreMesh(core_axis_name="c", subcore_axis_name="s")


@functools.partial(
    pl.kernel,
    out_type=(
        jax.ShapeDtypeStruct((_NW, _B), jnp.float32),
        jax.ShapeDtypeStruct((_NW, _B), jnp.int32),
    ),
    mesh=_mesh,
    compiler_params=pltpu.CompilerParams(needs_layout_passes=False),
    scratch_types=[
        pltpu.VMEM((_CHCLS, _B), jnp.float32),
        pltpu.VMEM((_CHCLS, _B), jnp.float32),
        pltpu.VMEM((_B,), jnp.float32),
        pltpu.VMEM((_B,), jnp.int32),
        pltpu.SemaphoreType.DMA,
        pltpu.SemaphoreType.DMA,
    ],
)
def _sc_argmax(predt_hbm, max_hbm, idx_hbm, buf0, buf1, vm_v, vi_v,
               sem0, sem1):
    wid = lax.axis_index("s") * _NC + lax.axis_index("c")
    start_tile = _T0 + (wid * (_NT - _T0 - _WT)) // (_NW - 1)
    cls0 = pl.multiple_of(start_tile * 8, 8)

    bufs = (buf0, buf1)
    sems = (sem0, sem1)

    def seg_slice(c):
        start = pl.multiple_of(cls0 + c * _CHCLS, 8)
        return predt_hbm.at[pl.ds(start, _CHCLS), :]

    def issue(c):
        pltpu.async_copy(seg_slice(c), bufs[c % 2], sems[c % 2])

    issue(0)
    best = [jnp.full((_L,), -jnp.inf, jnp.float32) for _ in range(_NG)]
    bcls = [jnp.zeros((_L,), jnp.int32) for _ in range(_NG)]
    for c in range(_NCH):
        if c + 1 < _NCH:
            issue(c + 1)
        pltpu.make_async_copy(seg_slice(c), bufs[c % 2], sems[c % 2]).wait()
        buf = bufs[c % 2]
        ccls0 = cls0 + c * _CHCLS

        def body(i, st, buf=buf, ccls0=ccls0):
            bb, bc = st
            clsv = jnp.full((_L,), ccls0 + i, jnp.int32)
            nb, nc2 = [], []
            for g in range(_NG):
                v = buf[i, pl.ds(g * _L, _L)]
                m = v > bb[g]
                nb.append(jnp.where(m, v, bb[g]))
                nc2.append(jnp.where(m, clsv, bc[g]))
            return nb, nc2

        best, bcls = plsc.parallel_loop(
            0, _CHCLS, unroll=2, carry=(best, bcls))(body)

    for g in range(_NG):
        vm_v[pl.ds(g * _L, _L)] = best[g]
        vi_v[pl.ds(g * _L, _L)] = bcls[g]
    pltpu.sync_copy(vm_v, max_hbm.at[wid])
    pltpu.sync_copy(vi_v, idx_hbm.at[wid])


def _tc_body(x_ref, m_ref, i_ref, am, ai, ab, iota_s):
    g = pl.program_id(0)

    @pl.when(g == 0)
    def _():
        am[...] = jnp.full((1, _B), -jnp.inf, jnp.float32)
        ai[...] = jnp.full((1, _B), _IMAX, jnp.int32)
        ab[...] = jnp.zeros((1, _B), jnp.int32)
        iota_s[...] = lax.broadcasted_iota(jnp.int32, (_BKTC, _B), 0)

    x = x_ref[...]
    bm = jnp.max(x, axis=0, keepdims=True)
    bi = jnp.min(jnp.where(x == bm, iota_s[...], _IMAX), axis=0,
                 keepdims=True)
    pm = am[...]
    new = bm > pm
    am[...] = jnp.where(new, bm, pm)
    ai[...] = jnp.where(new, bi, ai[...])
    ab[...] = jnp.where(new, jnp.full((1, _B), g, jnp.int32), ab[...])

    @pl.when(g == _GTC - 1)
    def _():
        m_ref[...] = am[...]
        i_ref[...] = ab[...] * _BKTC + ai[...]


_tc_argmax = pl.pallas_call(
    _tc_body,
    grid=(_GTC,),
    in_specs=[pl.BlockSpec((_BKTC, _B), lambda g: (g, 0))],
    out_specs=(
        pl.BlockSpec((1, _B), lambda g: (0, 0)),
        pl.BlockSpec((1, _B), lambda g: (0, 0)),
    ),
    out_shape=(
        jax.ShapeDtypeStruct((1, _B), jnp.float32),
        jax.ShapeDtypeStruct((1, _B), jnp.int32),
    ),
    scratch_shapes=[
        pltpu.VMEM((1, _B), jnp.float32),
        pltpu.VMEM((1, _B), jnp.int32),
        pltpu.VMEM((1, _B), jnp.int32),
        pltpu.VMEM((_BKTC, _B), jnp.int32),
    ],
)


def _merge_body(scm_ref, sci_ref, tcm_ref, tci_ref, tgt_ref, out_ref):
    m = scm_ref[...]
    i = sci_ref[...]
    tm = tcm_ref[...]
    ti = tci_ref[...]
    t = tgt_ref[...]
    rm = jnp.maximum(jnp.max(m, axis=0), tm[0])
    wi_sc = jnp.min(jnp.where(m == rm[None, :], i, _IMAX), axis=0)
    wi_tc = jnp.where(tm[0] == rm, ti[0], _IMAX)
    wi = jnp.minimum(wi_sc, wi_tc)
    cnt = jnp.sum(jnp.where(wi == t, 1.0, 0.0).astype(jnp.float32))
    out_ref[0] = cnt * (100.0 / _B)


_merge = pl.pallas_call(
    _merge_body,
    out_shape=jax.ShapeDtypeStruct((1,), jnp.float32),
    out_specs=pl.BlockSpec(memory_space=pltpu.SMEM),
)


def kernel(pred, target):
    predt = pred.T
    sc_m, sc_i = _sc_argmax(predt)
    tc_m, tc_i = _tc_argmax(predt)
    res = _merge(sc_m, sc_i, tc_m, tc_i, target)
    return (res,)


# speedup vs baseline: 1.0143x; 1.0143x over previous
import functools

import jax
import jax.numpy as jnp
from jax import lax
from jax.experimental import pallas as pl
from jax.experimental.pallas import tpu as pltpu
from jax.experimental.pallas import tpu_sc as plsc

_B = 128
_V = 100000
_NC = 2
_NS = 16
_NW = _NC * _NS
_L = 16
_NG = _B // _L

_KTC = 40000
_BKTC = 2000
_GTC = _KTC // _BKTC

_T0 = _KTC // 8
_NT = _V // 8
_WT = 235
_CHT = 47
_CHCLS = _CHT * 8
_NCH = _WT // _CHT
_IMAX = 2**31 - 1

_mesh = plsc.VectorSubcoreMesh(core_axis_name="c", subcore_axis_name="s")


@functools.partial(
    pl.kernel,
    out_type=(
        jax.ShapeDtypeStruct((_NW, _B), jnp.float32),
        jax.ShapeDtypeStruct((_NW, _B), jnp.int32),
    ),
    mesh=_mesh,
    compiler_params=pltpu.CompilerParams(needs_layout_passes=False),
    scratch_types=[
        pltpu.VMEM((_CHCLS, _B), jnp.float32),
        pltpu.VMEM((_CHCLS, _B), jnp.float32),
        pltpu.VMEM((_B,), jnp.float32),
        pltpu.VMEM((_B,), jnp.int32),
        pltpu.SemaphoreType.DMA,
        pltpu.SemaphoreType.DMA,
    ],
)
def _sc_argmax(predt_hbm, max_hbm, idx_hbm, buf0, buf1, vm_v, vi_v,
               sem0, sem1):
    wid = lax.axis_index("s") * _NC + lax.axis_index("c")
    start_tile = _T0 + (wid * (_NT - _T0 - _WT)) // (_NW - 1)
    cls0 = pl.multiple_of(start_tile * 8, 8)

    bufs = (buf0, buf1)
    sems = (sem0, sem1)

    def seg_slice(c):
        start = pl.multiple_of(cls0 + c * _CHCLS, 8)
        return predt_hbm.at[pl.ds(start, _CHCLS), :]

    def issue(c):
        pltpu.async_copy(seg_slice(c), bufs[c % 2], sems[c % 2])

    issue(0)
    best = [jnp.full((_L,), -jnp.inf, jnp.float32) for _ in range(_NG)]
    bcls = [jnp.zeros((_L,), jnp.int32) for _ in range(_NG)]
    for c in range(_NCH):
        if c + 1 < _NCH:
            issue(c + 1)
        pltpu.make_async_copy(seg_slice(c), bufs[c % 2], sems[c % 2]).wait()
        buf = bufs[c % 2]
        ccls0 = cls0 + c * _CHCLS

        def body(i, st, buf=buf, ccls0=ccls0):
            bb, bc = st
            clsv = jnp.full((_L,), ccls0 + i, jnp.int32)
            nb, nc2 = [], []
            for g in range(_NG):
                v = buf[i, pl.ds(g * _L, _L)]
                m = v > bb[g]
                nb.append(jnp.where(m, v, bb[g]))
                nc2.append(jnp.where(m, clsv, bc[g]))
            return nb, nc2

        best, bcls = plsc.parallel_loop(
            0, _CHCLS, unroll=2, carry=(best, bcls))(body)

    for g in range(_NG):
        vm_v[pl.ds(g * _L, _L)] = best[g]
        vi_v[pl.ds(g * _L, _L)] = bcls[g]
    pltpu.sync_copy(vm_v, max_hbm.at[wid])
    pltpu.sync_copy(vi_v, idx_hbm.at[wid])


def _tc_body(x_ref, m_ref, i_ref, am, ai, ab, iota_s):
    g = pl.program_id(0)

    @pl.when(g == 0)
    def _():
        am[...] = jnp.full((1, _B), -jnp.inf, jnp.float32)
        ai[...] = jnp.full((1, _B), _IMAX, jnp.int32)
        ab[...] = jnp.zeros((1, _B), jnp.int32)
        iota_s[...] = lax.broadcasted_iota(jnp.int32, (_BKTC, _B), 0)

    x = x_ref[...]
    bm = jnp.max(x, axis=0, keepdims=True)
    bi = jnp.min(jnp.where(x == bm, iota_s[...], _IMAX), axis=0,
                 keepdims=True)
    pm = am[...]
    new = bm > pm
    am[...] = jnp.where(new, bm, pm)
    ai[...] = jnp.where(new, bi, ai[...])
    ab[...] = jnp.where(new, jnp.full((1, _B), g, jnp.int32), ab[...])

    @pl.when(g == _GTC - 1)
    def _():
        m_ref[...] = am[...]
        i_ref[...] = ab[...] * _BKTC + ai[...]


_tc_argmax = pl.pallas_call(
    _tc_body,
    grid=(_GTC,),
    in_specs=[pl.BlockSpec((_BKTC, _B), lambda g: (g, 0))],
    out_specs=(
        pl.BlockSpec((1, _B), lambda g: (0, 0)),
        pl.BlockSpec((1, _B), lambda g: (0, 0)),
    ),
    out_shape=(
        jax.ShapeDtypeStruct((1, _B), jnp.float32),
        jax.ShapeDtypeStruct((1, _B), jnp.int32),
    ),
    scratch_shapes=[
        pltpu.VMEM((1, _B), jnp.float32),
        pltpu.VMEM((1, _B), jnp.int32),
        pltpu.VMEM((1, _B), jnp.int32),
        pltpu.VMEM((_BKTC, _B), jnp.int32),
    ],
)


def _merge_body(scm_ref, sci_ref, tcm_ref, tci_ref, tgt_ref, out_ref):
    m = scm_ref[...]
    i = sci_ref[...]
    tm = tcm_ref[...]
    ti = tci_ref[...]
    t = tgt_ref[...]
    rm = jnp.maximum(jnp.max(m, axis=0), tm[0])
    wi_sc = jnp.min(jnp.where(m == rm[None, :], i, _IMAX), axis=0)
    wi_tc = jnp.where(tm[0] == rm, ti[0], _IMAX)
    wi = jnp.minimum(wi_sc, wi_tc)
    cnt = jnp.sum(jnp.where(wi == t, 1.0, 0.0).astype(jnp.float32))
    out_ref[0] = cnt * (100.0 / _B)


_merge = pl.pallas_call(
    _merge_body,
    out_shape=jax.ShapeDtypeStruct((1,), jnp.float32),
    out_specs=pl.BlockSpec(memory_space=pltpu.SMEM),
)


def kernel(pred, target):
    predt = pred.T
    sc_m, sc_i = _sc_argmax(predt)
    tc_m, tc_i = _tc_argmax(predt)
    res = _merge(sc_m, sc_i, tc_m, tc_i, target)
    return (res,)
